# baseline (speedup 1.0000x reference)
import functools

import jax
import jax.numpy as jnp
from jax import lax
from jax.experimental import pallas as pl
from jax.experimental.pallas import tpu as pltpu
from jax.experimental.pallas import tpu_sc as plsc

_B, _D, _L = 8, 64, 1024
_K = 8192
_N = _B * _L
_COMMIT = 0.25

_NT = 2048
_BT = _NT // _L
_CK = 256
_GRID_N = _N // _NT
_KT = 2048


def _cb_body(cb_ref, cn_ref):
    t = cb_ref[...]
    ss = jnp.sum(t * t, axis=1, keepdims=True)
    den = jnp.maximum(jnp.sqrt(ss), 1e-12)
    cn_ref[...] = jnp.concatenate(
        [t / den, jnp.zeros((t.shape[0], 128 - _D), jnp.float32)], axis=1)


def _cb_normalize(codebook, interpret=False):
    return pl.pallas_call(
        _cb_body,
        grid=(_K // _KT,),
        in_specs=[pl.BlockSpec((_KT, _D), lambda i: (i, 0))],
        out_specs=[pl.BlockSpec((_KT, 128), lambda i: (i, 0))],
        out_shape=[jax.ShapeDtypeStruct((_K, 128), jnp.float32)],
        compiler_params=pltpu.CompilerParams(
            dimension_semantics=("parallel",)),
        interpret=interpret,
    )(codebook)[0]


def _vq_body(x_ref, cn_ref, idx_ref, commit_ref):
    xt = jnp.transpose(x_ref[...], (0, 2, 1)).reshape(_NT, _D)
    ss = jnp.sum(xt * xt, axis=1, keepdims=True)
    nrm = jnp.sqrt(ss)
    den = jnp.maximum(nrm, 1e-12)
    fn = jnp.concatenate(
        [xt / den, jnp.zeros((_NT, 128 - _D), jnp.float32)], axis=1)

    m = jnp.full((_NT, 128), -jnp.inf, jnp.float32)
    aux = jnp.full((_NT, 128), -jnp.inf, jnp.float32)
    kb = jnp.zeros((_NT, 128), jnp.int32)
    for c in range(_K // _CK):
        sim = lax.dot_general(
            fn, cn_ref[c * _CK:(c + 1) * _CK, :],
            (((1,), (1,)), ((), ())),
            preferred_element_type=jnp.float32,
        )
        s_a = sim[:, 0:128]
        s_b = sim[:, 128:256]
        mm = jnp.maximum(s_a, s_b)
        upd = mm > m
        m = jnp.where(upd, mm, m)
        kb = jnp.where(upd, jnp.int32(c), kb)
        aux = jnp.where(upd, s_a, aux)

    row_max = jnp.max(m, axis=1, keepdims=True)
    kglob = (kb * _CK
             + jnp.where(aux == m, jnp.int32(0), jnp.int32(128))
             + lax.broadcasted_iota(jnp.int32, m.shape, 1))
    cand = jnp.where(m == row_max, kglob, jnp.int32(1 << 30))
    idx_ref[...] = jnp.min(cand, axis=1)

    part = jnp.sum(nrm * nrm - 2.0 * nrm * row_max + 1.0, keepdims=True)
    commit_ref[...] = part.reshape(1, 1, 1) * (_COMMIT / (_N * _D))


def _vq_argmax(x, cn_pad, interpret=False):
    return pl.pallas_call(
        _vq_body,
        grid=(_GRID_N,),
        in_specs=[
            pl.BlockSpec((_BT, _D, _L), lambda n: (n, 0, 0)),
            pl.BlockSpec((_K, 128), lambda n: (0, 0)),
        ],
        out_specs=[
            pl.BlockSpec((_NT,), lambda n: (n,)),
            pl.BlockSpec((1, 1, 1), lambda n: (n, 0, 0)),
        ],
        out_shape=[
            jax.ShapeDtypeStruct((_N,), jnp.int32),
            jax.ShapeDtypeStruct((_GRID_N, 1, 1), jnp.float32),
        ],
        compiler_params=pltpu.CompilerParams(
            dimension_semantics=("parallel",)),
        interpret=interpret,
    )(x, cn_pad)


def _gather(cn_pad, idx):
    info = plsc.get_sparse_core_info()
    nc, ns = info.num_cores, info.num_subcores
    nw = nc * ns
    b_per_w = _N // nw
    mesh = plsc.VectorSubcoreMesh(core_axis_name="c", subcore_axis_name="s")

    @functools.partial(
        pl.kernel,
        out_type=jax.ShapeDtypeStruct((_N, 128), jnp.float32),
        mesh=mesh,
        scratch_types=[
            pltpu.VMEM((b_per_w,), jnp.int32),
            pltpu.VMEM((b_per_w, 128), jnp.float32),
            pltpu.SemaphoreType.DMA,
        ],
    )
    def gather_k(table_hbm, idx_hbm, out_hbm, idx_v, rows_v, sem):
        wid = lax.axis_index("s") * nc + lax.axis_index("c")
        base = wid * b_per_w
        pltpu.sync_copy(idx_hbm.at[pl.ds(base, b_per_w)], idx_v)
        pltpu.async_copy(table_hbm.at[idx_v], rows_v, sem).wait()
        pltpu.sync_copy(rows_v, out_hbm.at[pl.ds(base, b_per_w)])

    return gather_k(cn_pad, idx)


def kernel(x, codebook):
    cn_pad = _cb_normalize(codebook)
    idx, commit = _vq_argmax(x, cn_pad)
    q_flat = _gather(cn_pad, idx)
    q = jnp.transpose(q_flat.reshape(_B, _L, 128)[:, :, :_D], (0, 2, 1))
    return q, jnp.sum(commit)


# speedup vs baseline: 1.1667x; 1.1667x over previous
import functools

import jax
import jax.numpy as jnp
from jax import lax
from jax.experimental import pallas as pl
from jax.experimental.pallas import tpu as pltpu
from jax.experimental.pallas import tpu_sc as plsc

_B, _D, _L = 8, 64, 1024
_K = 8192
_N = _B * _L
_COMMIT = 0.25

_NT = 2048
_BT = _NT // _L
_CK = 256
_GRID_N = _N // _NT


def _vq_body(x_ref, cb_ref, idx_ref, commit_ref, cn_out_ref, cn_sc):
    n = pl.program_id(0)

    @pl.when(n == 0)
    def _():
        t = cb_ref[...]
        ss = jnp.sum(t * t, axis=1, keepdims=True)
        den = jnp.maximum(jnp.sqrt(ss), 1e-12)
        cn = jnp.concatenate(
            [t / den, jnp.zeros((_K, 128 - _D), jnp.float32)], axis=1)
        cn_sc[...] = cn
        cn_out_ref[...] = cn

    xt = jnp.transpose(x_ref[...], (0, 2, 1)).reshape(_NT, _D)
    ss = jnp.sum(xt * xt, axis=1, keepdims=True)
    nrm = jnp.sqrt(ss)
    den = jnp.maximum(nrm, 1e-12)
    fn = jnp.concatenate(
        [xt / den, jnp.zeros((_NT, 128 - _D), jnp.float32)], axis=1)

    m = jnp.full((_NT, 128), -jnp.inf, jnp.float32)
    aux = jnp.full((_NT, 128), -jnp.inf, jnp.float32)
    kb = jnp.zeros((_NT, 128), jnp.int32)
    for c in range(_K // _CK):
        sim = lax.dot_general(
            fn, cn_sc[c * _CK:(c + 1) * _CK, :],
            (((1,), (1,)), ((), ())),
            preferred_element_type=jnp.float32,
        )
        s_a = sim[:, 0:128]
        s_b = sim[:, 128:256]
        mm = jnp.maximum(s_a, s_b)
        upd = mm > m
        m = jnp.where(upd, mm, m)
        kb = jnp.where(upd, jnp.int32(c), kb)
        aux = jnp.where(upd, s_a, aux)

    row_max = jnp.max(m, axis=1, keepdims=True)
    kglob = (kb * _CK
             + jnp.where(aux == m, jnp.int32(0), jnp.int32(128))
             + lax.broadcasted_iota(jnp.int32, m.shape, 1))
    cand = jnp.where(m == row_max, kglob, jnp.int32(1 << 30))
    idx_ref[...] = jnp.min(cand, axis=1).reshape(_NT // 256, 256)

    part = jnp.sum(nrm * nrm - 2.0 * nrm * row_max + 1.0,
                   keepdims=True)[:, :1]

    @pl.when(n == 0)
    def _():
        commit_ref[...] = jnp.zeros_like(commit_ref[...])
    commit_ref[...] += part * (_COMMIT / (_N * _D))


def _vq_argmax(x, codebook, interpret=False):
    return pl.pallas_call(
        _vq_body,
        grid=(_GRID_N,),
        in_specs=[
            pl.BlockSpec((_BT, _D, _L), lambda n: (n, 0, 0)),
            pl.BlockSpec((_K, _D), lambda n: (0, 0)),
        ],
        out_specs=[
            pl.BlockSpec((_NT // 256, 256), lambda n: (n, 0)),
            pl.BlockSpec((1, 1), lambda n: (0, 0)),
            pl.BlockSpec((_K, 128), lambda n: (0, 0)),
        ],
        out_shape=[
            jax.ShapeDtypeStruct((_N // 256, 256), jnp.int32),
            jax.ShapeDtypeStruct((1, 1), jnp.float32),
            jax.ShapeDtypeStruct((_K, 128), jnp.float32),
        ],
        scratch_shapes=[
            pltpu.VMEM((_K, 128), jnp.float32),
        ],
        interpret=interpret,
    )(x, codebook)


def _gather(cn_pad, idx):
    info = plsc.get_sparse_core_info()
    nc, ns = info.num_cores, info.num_subcores
    nw = nc * ns
    b_per_w = _N // nw
    mesh = plsc.VectorSubcoreMesh(core_axis_name="c", subcore_axis_name="s")

    @functools.partial(
        pl.kernel,
        out_type=jax.ShapeDtypeStruct((_N, 128), jnp.float32),
        mesh=mesh,
        scratch_types=[
            pltpu.VMEM((b_per_w,), jnp.int32),
            pltpu.VMEM((b_per_w, 128), jnp.float32),
            pltpu.SemaphoreType.DMA,
        ],
    )
    def gather_k(table_hbm, idx_hbm, out_hbm, idx_v, rows_v, sem):
        wid = lax.axis_index("s") * nc + lax.axis_index("c")
        base = wid * b_per_w
        pltpu.sync_copy(idx_hbm.at[wid], idx_v)
        pltpu.async_copy(table_hbm.at[idx_v], rows_v, sem).wait()
        pltpu.sync_copy(rows_v, out_hbm.at[pl.ds(base, b_per_w)])

    return gather_k(cn_pad, idx)


def kernel(x, codebook):
    idx, commit, cn_pad = _vq_argmax(x, codebook)
    q_flat = _gather(cn_pad, idx)
    q = jnp.transpose(q_flat.reshape(_B, _L, 128)[:, :, :_D], (0, 2, 1))
    return q, commit.reshape(())
